# baseline (speedup 1.0000x reference)
import functools

import jax
import jax.numpy as jnp
from jax import lax
from jax.experimental import pallas as pl
from jax.experimental.pallas import tpu as pltpu
from jax.experimental.pallas import tpu_sc as plsc

_N = 10000
_E = 320000
_D = 128
_C = 40
_CP = 128

_CH = 128
_NSC = 2
_NT = 16
_K = 80
_EPAD = _NSC * _NT * _K * _CH - _E
_RT = 632
_NP = _NT * _RT

_mesh = plsc.VectorSubcoreMesh(core_axis_name="c", subcore_axis_name="s")


def _deg_body(dst_hbm, ones_hbm, z_hbm, out_hbm, dsti, ones_v, acc):
    c = lax.axis_index("c")
    s = lax.axis_index("s")
    pltpu.sync_copy(ones_hbm, ones_v)
    pltpu.sync_copy(dst_hbm.at[c, s], dsti)
    pltpu.sync_copy(z_hbm.at[pl.ds(s * _RT, _RT)], acc.at[pl.ds(s * _RT, _RT)])
    plsc.subcore_barrier()

    def body(k, carry):
        pltpu.sync_copy(ones_v, acc.at[dsti.at[k]], add=True)
        return carry

    lax.fori_loop(0, _K, body, 0)
    plsc.subcore_barrier()
    pltpu.sync_copy(acc.at[pl.ds(s * _RT, _RT)],
                    out_hbm.at[c, pl.ds(s * _RT, _RT)])


_deg_kernel = functools.partial(
    pl.kernel,
    out_type=jax.ShapeDtypeStruct((_NSC, _NP, _D), jnp.float32),
    mesh=_mesh,
    scratch_types=[
        pltpu.VMEM((_K, _CH), jnp.int32),
        pltpu.VMEM((_CH, _D), jnp.float32),
        pltpu.VMEM_SHARED((_NP, _D), jnp.float32),
    ],
)(_deg_body)


def _make_agg(d):
    def body(src_hbm, dst_hbm, table_hbm, z_hbm, out_hbm, srci, dsti, row,
             acc):
        c = lax.axis_index("c")
        s = lax.axis_index("s")
        pltpu.sync_copy(src_hbm.at[c, s], srci)
        pltpu.sync_copy(dst_hbm.at[c, s], dsti)
        pltpu.sync_copy(z_hbm.at[pl.ds(s * _RT, _RT)],
                        acc.at[pl.ds(s * _RT, _RT)])
        plsc.subcore_barrier()

        def chunk(k, carry):
            pltpu.sync_copy(table_hbm.at[srci.at[k]], row)
            pltpu.sync_copy(row, acc.at[dsti.at[k]], add=True)
            return carry

        lax.fori_loop(0, _K, chunk, 0)
        plsc.subcore_barrier()
        pltpu.sync_copy(acc.at[pl.ds(s * _RT, _RT)],
                        out_hbm.at[c, pl.ds(s * _RT, _RT)])

    return pl.kernel(
        body,
        out_type=jax.ShapeDtypeStruct((_NSC, _NP, d), jnp.float32),
        mesh=_mesh,
        scratch_types=[
            pltpu.VMEM((_K, _CH), jnp.int32),
            pltpu.VMEM((_K, _CH), jnp.int32),
            pltpu.VMEM((_CH, d), jnp.float32),
            pltpu.VMEM_SHARED((_NP, d), jnp.float32),
        ],
    )


_agg128 = _make_agg(_D)

_R = 1000


def _dinv_of(cnt_ref):
    deg = cnt_ref[0, :, 0:1] + cnt_ref[1, :, 0:1] + 1.0
    return lax.rsqrt(deg)


def _tc1_body(x_ref, w_ref, cnt_ref, o_ref):
    dinv = _dinv_of(cnt_ref)
    o_ref[...] = jnp.dot(x_ref[...], w_ref[...],
                         preferred_element_type=jnp.float32) * dinv


def _tc_mid_body(p_ref, g_ref, cnt_ref, b_ref, w_ref, o_ref):
    dinv = _dinv_of(cnt_ref)
    svec = p_ref[0] + p_ref[1] + g_ref[...]
    h = jnp.maximum(svec * dinv + b_ref[...][None, :], 0.0)
    o_ref[...] = jnp.dot(h, w_ref[...],
                         preferred_element_type=jnp.float32) * dinv


def _tc4_body(p_ref, g_ref, cnt_ref, b_ref, o_ref):
    dinv = _dinv_of(cnt_ref)
    z = (p_ref[0] + p_ref[1] + g_ref[...]) * dinv + b_ref[...][None, :]
    col = lax.broadcasted_iota(jnp.int32, z.shape, 1)
    zm = jnp.where(col < _C, z, -jnp.inf)
    m = jnp.max(zm, axis=1, keepdims=True)
    ez = jnp.where(col < _C, jnp.exp(z - m), 0.0)
    lse = jnp.log(jnp.sum(ez, axis=1, keepdims=True))
    o_ref[...] = z - m - lse


def _row_spec(w):
    return pl.BlockSpec((_R, w), lambda i: (i, 0))


def _cnt_spec():
    return pl.BlockSpec((_NSC, _R, _D), lambda i: (0, i, 0))


def _full_spec(shape):
    nd = len(shape)
    return pl.BlockSpec(shape, lambda i: (0,) * nd)


def kernel(x, edge_index, W1, b1, W2, b2, W3, b3):
    src = jnp.concatenate(
        [edge_index[0], jnp.zeros((_EPAD,), jnp.int32)]
    ).reshape(_NSC, _NT, _K, _CH)
    dst = jnp.concatenate(
        [edge_index[1], jnp.full((_EPAD,), _N, jnp.int32)]
    ).reshape(_NSC, _NT, _K, _CH)

    ones128 = jnp.ones((_CH, _D), jnp.float32)
    z128 = jnp.zeros((_NP, _D), jnp.float32)
    W3p = jnp.concatenate([W3, jnp.zeros((_D, _CP - _C), jnp.float32)], axis=1)
    b3p = jnp.concatenate([b3, jnp.zeros((_CP - _C,), jnp.float32)])

    counts = _deg_kernel(dst, ones128, z128)

    g1 = pl.pallas_call(
        _tc1_body,
        grid=(_N // _R,),
        in_specs=[_row_spec(_D), _full_spec((_D, _D)), _cnt_spec()],
        out_specs=_row_spec(_D),
        out_shape=jax.ShapeDtypeStruct((_N, _D), jnp.float32),
    )(x, W1, counts)

    p1 = _agg128(src, dst, g1, z128)

    def mid(p, g, b, w, wout):
        return pl.pallas_call(
            _tc_mid_body,
            grid=(_N // _R,),
            in_specs=[
                pl.BlockSpec((_NSC, _R, _D), lambda i: (0, i, 0)),
                _row_spec(_D),
                _cnt_spec(),
                _full_spec((_D,)),
                _full_spec((_D, wout)),
            ],
            out_specs=_row_spec(wout),
            out_shape=jax.ShapeDtypeStruct((_N, wout), jnp.float32),
        )(p, g, counts, b, w)

    g2 = mid(p1, g1, b1, W2, _D)
    p2 = _agg128(src, dst, g2, z128)
    g3 = mid(p2, g2, b2, W3p, _CP)
    p3 = _agg128(src, dst, g3, z128)

    out = pl.pallas_call(
        _tc4_body,
        grid=(_N // _R,),
        in_specs=[
            pl.BlockSpec((_NSC, _R, _CP), lambda i: (0, i, 0)),
            _row_spec(_CP),
            _cnt_spec(),
            _full_spec((_CP,)),
        ],
        out_specs=_row_spec(_CP),
        out_shape=jax.ShapeDtypeStruct((_N, _CP), jnp.float32),
    )(p3, g3, counts, b3p)

    return out[:, :_C]


# speedup vs baseline: 1.1137x; 1.1137x over previous
import functools

import jax
import jax.numpy as jnp
from jax import lax
from jax.experimental import pallas as pl
from jax.experimental.pallas import tpu as pltpu
from jax.experimental.pallas import tpu_sc as plsc

_N = 10000
_E = 320000
_D = 128
_C = 40
_CP = 128

_CH = 128
_NSC = 2
_NT = 16
_K = 80
_KS = 40
_NSL = _K // _KS
_EPAD = _NSC * _NT * _K * _CH - _E
_RT = 632
_NP = _NT * _RT

_mesh = plsc.VectorSubcoreMesh(core_axis_name="c", subcore_axis_name="s")


def _deg_body(dst_hbm, ones_hbm, z_hbm, out_hbm, dsti, ones_v, acc):
    c = lax.axis_index("c")
    s = lax.axis_index("s")
    pltpu.sync_copy(ones_hbm, ones_v)
    pltpu.sync_copy(dst_hbm.at[c, s], dsti)
    pltpu.sync_copy(z_hbm.at[pl.ds(s * _RT, _RT)], acc.at[pl.ds(s * _RT, _RT)])
    plsc.subcore_barrier()

    def body(k, carry):
        pltpu.sync_copy(ones_v, acc.at[dsti.at[k]], add=True)
        return carry

    lax.fori_loop(0, _K, body, 0)
    plsc.subcore_barrier()
    pltpu.sync_copy(acc.at[pl.ds(s * _RT, _RT)],
                    out_hbm.at[c, pl.ds(s * _RT, _RT)])


_deg_kernel = functools.partial(
    pl.kernel,
    out_type=jax.ShapeDtypeStruct((_NSC, _NP, _D), jnp.float32),
    mesh=_mesh,
    scratch_types=[
        pltpu.VMEM((_K, _CH), jnp.int32),
        pltpu.VMEM((_CH, _D), jnp.float32),
        pltpu.VMEM_SHARED((_NP, _D), jnp.float32),
    ],
)(_deg_body)


def _make_agg(d):
    def body(src_hbm, dst_hbm, table_hbm, z_hbm, out_hbm, srci, dsti, r0, r1,
             acc, g0, g1):
        c = lax.axis_index("c")
        s = lax.axis_index("s")
        rows = [r0, r1]
        sems = [g0, g1]
        pltpu.sync_copy(z_hbm.at[pl.ds(s * _RT, _RT)],
                        acc.at[pl.ds(s * _RT, _RT)])
        plsc.subcore_barrier()

        for sl in range(_NSL):
            pltpu.sync_copy(src_hbm.at[c, s, pl.ds(sl * _KS, _KS)], srci)
            pltpu.sync_copy(dst_hbm.at[c, s, pl.ds(sl * _KS, _KS)], dsti)
            pltpu.async_copy(table_hbm.at[srci.at[0]], r0, g0)
            pltpu.async_copy(table_hbm.at[srci.at[1]], r1, g1)

            def chunk2(k2, carry):
                for p in range(2):
                    k = k2 * 2 + p
                    pltpu.make_async_copy(table_hbm.at[srci.at[k]], rows[p],
                                          sems[p]).wait()
                    pltpu.sync_copy(rows[p], acc.at[dsti.at[k]], add=True)

                    @pl.when(k + 2 < _KS)
                    def _issue():
                        pltpu.async_copy(table_hbm.at[srci.at[k + 2]],
                                         rows[p], sems[p])

                return carry

            lax.fori_loop(0, _KS // 2, chunk2, 0)

        plsc.subcore_barrier()
        pltpu.sync_copy(acc.at[pl.ds(s * _RT, _RT)],
                        out_hbm.at[c, pl.ds(s * _RT, _RT)])

    return pl.kernel(
        body,
        out_type=jax.ShapeDtypeStruct((_NSC, _NP, d), jnp.float32),
        mesh=_mesh,
        scratch_types=[
            pltpu.VMEM((_KS, _CH), jnp.int32),
            pltpu.VMEM((_KS, _CH), jnp.int32),
            pltpu.VMEM((_CH, d), jnp.float32),
            pltpu.VMEM((_CH, d), jnp.float32),
            pltpu.VMEM_SHARED((_NP, d), jnp.float32),
            pltpu.SemaphoreType.DMA,
            pltpu.SemaphoreType.DMA,
        ],
    )


_agg128 = _make_agg(_D)

_R = 1000


def _dinv_of(cnt_ref):
    deg = cnt_ref[0, :, 0:1] + cnt_ref[1, :, 0:1] + 1.0
    return lax.rsqrt(deg)


def _tc1_body(x_ref, w_ref, cnt_ref, o_ref):
    dinv = _dinv_of(cnt_ref)
    o_ref[...] = jnp.dot(x_ref[...], w_ref[...],
                         preferred_element_type=jnp.float32) * dinv


def _tc_mid_body(p_ref, g_ref, cnt_ref, b_ref, w_ref, o_ref):
    dinv = _dinv_of(cnt_ref)
    svec = p_ref[0] + p_ref[1] + g_ref[...]
    h = jnp.maximum(svec * dinv + b_ref[...][None, :], 0.0)
    o_ref[...] = jnp.dot(h, w_ref[...],
                         preferred_element_type=jnp.float32) * dinv


def _tc4_body(p_ref, g_ref, cnt_ref, b_ref, o_ref):
    dinv = _dinv_of(cnt_ref)
    z = (p_ref[0] + p_ref[1] + g_ref[...]) * dinv + b_ref[...][None, :]
    col = lax.broadcasted_iota(jnp.int32, z.shape, 1)
    zm = jnp.where(col < _C, z, -jnp.inf)
    m = jnp.max(zm, axis=1, keepdims=True)
    ez = jnp.where(col < _C, jnp.exp(z - m), 0.0)
    lse = jnp.log(jnp.sum(ez, axis=1, keepdims=True))
    o_ref[...] = z - m - lse


def _row_spec(w):
    return pl.BlockSpec((_R, w), lambda i: (i, 0))


def _cnt_spec():
    return pl.BlockSpec((_NSC, _R, _D), lambda i: (0, i, 0))


def _full_spec(shape):
    nd = len(shape)
    return pl.BlockSpec(shape, lambda i: (0,) * nd)


def kernel(x, edge_index, W1, b1, W2, b2, W3, b3):
    src = jnp.concatenate(
        [edge_index[0], jnp.zeros((_EPAD,), jnp.int32)]
    ).reshape(_NSC, _NT, _K, _CH)
    dst = jnp.concatenate(
        [edge_index[1], jnp.full((_EPAD,), _N, jnp.int32)]
    ).reshape(_NSC, _NT, _K, _CH)

    ones128 = jnp.ones((_CH, _D), jnp.float32)
    z128 = jnp.zeros((_NP, _D), jnp.float32)
    W3p = jnp.concatenate([W3, jnp.zeros((_D, _CP - _C), jnp.float32)], axis=1)
    b3p = jnp.concatenate([b3, jnp.zeros((_CP - _C,), jnp.float32)])

    counts = _deg_kernel(dst, ones128, z128)

    g1 = pl.pallas_call(
        _tc1_body,
        grid=(_N // _R,),
        in_specs=[_row_spec(_D), _full_spec((_D, _D)), _cnt_spec()],
        out_specs=_row_spec(_D),
        out_shape=jax.ShapeDtypeStruct((_N, _D), jnp.float32),
    )(x, W1, counts)

    p1 = _agg128(src, dst, g1, z128)

    def mid(p, g, b, w, wout):
        return pl.pallas_call(
            _tc_mid_body,
            grid=(_N // _R,),
            in_specs=[
                pl.BlockSpec((_NSC, _R, _D), lambda i: (0, i, 0)),
                _row_spec(_D),
                _cnt_spec(),
                _full_spec((_D,)),
                _full_spec((_D, wout)),
            ],
            out_specs=_row_spec(wout),
            out_shape=jax.ShapeDtypeStruct((_N, wout), jnp.float32),
        )(p, g, counts, b, w)

    g2 = mid(p1, g1, b1, W2, _D)
    p2 = _agg128(src, dst, g2, z128)
    g3 = mid(p2, g2, b2, W3p, _CP)
    p3 = _agg128(src, dst, g3, z128)

    out = pl.pallas_call(
        _tc4_body,
        grid=(_N // _R,),
        in_specs=[
            pl.BlockSpec((_NSC, _R, _CP), lambda i: (0, i, 0)),
            _row_spec(_CP),
            _cnt_spec(),
            _full_spec((_CP,)),
        ],
        out_specs=_row_spec(_CP),
        out_shape=jax.ShapeDtypeStruct((_N, _CP), jnp.float32),
    )(p3, g3, counts, b3p)

    return out[:, :_C]


# speedup vs baseline: 1.4801x; 1.3289x over previous
import functools

import jax
import jax.numpy as jnp
from jax import lax
from jax.experimental import pallas as pl
from jax.experimental.pallas import tpu as pltpu
from jax.experimental.pallas import tpu_sc as plsc

_N = 10000
_E = 320000
_D = 128
_C = 40
_CP = 128

_CH = 128
_NSC = 2
_NT = 16
_K = 79
_EPAD = _NSC * _NT * _K * _CH - _E
_RT = 632
_NP = _NT * _RT

_mesh = plsc.VectorSubcoreMesh(core_axis_name="c", subcore_axis_name="s")


def _deg_body(dst_hbm, ones_hbm, z_hbm, out_hbm, dsti, ones_v, acc):
    c = lax.axis_index("c")
    s = lax.axis_index("s")
    pltpu.sync_copy(ones_hbm, ones_v)
    pltpu.sync_copy(dst_hbm.at[c, s], dsti)
    pltpu.sync_copy(z_hbm.at[pl.ds(s * _RT, _RT)], acc.at[pl.ds(s * _RT, _RT)])
    plsc.subcore_barrier()

    def body(k, carry):
        pltpu.sync_copy(ones_v, acc.at[dsti.at[k]], add=True)
        return carry

    lax.fori_loop(0, _K, body, 0)
    plsc.subcore_barrier()
    pltpu.sync_copy(acc.at[pl.ds(s * _RT, _RT)],
                    out_hbm.at[c, pl.ds(s * _RT, _RT)])


_deg_kernel = functools.partial(
    pl.kernel,
    out_type=jax.ShapeDtypeStruct((_NSC, _NP, _D), jnp.float32),
    mesh=_mesh,
    scratch_types=[
        pltpu.VMEM((_K, _CH), jnp.int32),
        pltpu.VMEM((_CH, _D), jnp.float32),
        pltpu.VMEM_SHARED((_NP, _D), jnp.float32),
    ],
)(_deg_body)


def _make_agg(d):
    def body(src_hbm, dst_hbm, table_hbm, z_hbm, out_hbm, srci, dsti, row,
             acc):
        c = lax.axis_index("c")
        s = lax.axis_index("s")
        pltpu.sync_copy(src_hbm.at[c, s], srci)
        pltpu.sync_copy(dst_hbm.at[c, s], dsti)
        pltpu.sync_copy(z_hbm.at[pl.ds(s * _RT, _RT)],
                        acc.at[pl.ds(s * _RT, _RT)])
        plsc.subcore_barrier()

        def chunk(k, carry):
            pltpu.sync_copy(table_hbm.at[srci.at[k]], row)
            pltpu.sync_copy(row, acc.at[dsti.at[k]], add=True)
            return carry

        lax.fori_loop(0, _K, chunk, 0)
        plsc.subcore_barrier()
        pltpu.sync_copy(acc.at[pl.ds(s * _RT, _RT)],
                        out_hbm.at[c, pl.ds(s * _RT, _RT)])

    return pl.kernel(
        body,
        out_type=jax.ShapeDtypeStruct((_NSC, _NP, d), jnp.float32),
        mesh=_mesh,
        scratch_types=[
            pltpu.VMEM((_K, _CH), jnp.int32),
            pltpu.VMEM((_K, _CH), jnp.int32),
            pltpu.VMEM((_CH, d), jnp.float32),
            pltpu.VMEM_SHARED((_NP, d), jnp.float32),
        ],
    )


_agg128 = _make_agg(_D)

_R = 1000


def _dinv_of(cnt_ref):
    deg = cnt_ref[0, :, 0:1] + cnt_ref[1, :, 0:1] + 1.0
    return lax.rsqrt(deg)


def _tc1_body(x_ref, w_ref, cnt_ref, o_ref):
    dinv = _dinv_of(cnt_ref)
    o_ref[...] = jnp.dot(x_ref[...], w_ref[...],
                         preferred_element_type=jnp.float32) * dinv


def _tc_mid_body(p_ref, g_ref, cnt_ref, b_ref, w_ref, o_ref):
    dinv = _dinv_of(cnt_ref)
    svec = p_ref[0] + p_ref[1] + g_ref[...]
    h = jnp.maximum(svec * dinv + b_ref[...][None, :], 0.0)
    o_ref[...] = jnp.dot(h, w_ref[...],
                         preferred_element_type=jnp.float32) * dinv


def _tc4_body(p_ref, g_ref, cnt_ref, b_ref, o_ref):
    dinv = _dinv_of(cnt_ref)
    z = (p_ref[0] + p_ref[1] + g_ref[...]) * dinv + b_ref[...][None, :]
    col = lax.broadcasted_iota(jnp.int32, z.shape, 1)
    zm = jnp.where(col < _C, z, -jnp.inf)
    m = jnp.max(zm, axis=1, keepdims=True)
    ez = jnp.where(col < _C, jnp.exp(z - m), 0.0)
    lse = jnp.log(jnp.sum(ez, axis=1, keepdims=True))
    o_ref[...] = z - m - lse


def _row_spec(w):
    return pl.BlockSpec((_R, w), lambda i: (i, 0))


def _cnt_spec():
    return pl.BlockSpec((_NSC, _R, _D), lambda i: (0, i, 0))


def _full_spec(shape):
    nd = len(shape)
    return pl.BlockSpec(shape, lambda i: (0,) * nd)


def kernel(x, edge_index, W1, b1, W2, b2, W3, b3):
    src = jnp.concatenate(
        [edge_index[0], jnp.zeros((_EPAD,), jnp.int32)]
    ).reshape(_NSC, _NT, _K, _CH)
    dst = jnp.concatenate(
        [edge_index[1], jnp.full((_EPAD,), _N, jnp.int32)]
    ).reshape(_NSC, _NT, _K, _CH)

    ones128 = jnp.ones((_CH, _D), jnp.float32)
    z128 = jnp.zeros((_NP, _D), jnp.float32)
    W3p = jnp.concatenate([W3, jnp.zeros((_D, _CP - _C), jnp.float32)], axis=1)
    b3p = jnp.concatenate([b3, jnp.zeros((_CP - _C,), jnp.float32)])

    counts = _deg_kernel(dst, ones128, z128)

    g1 = pl.pallas_call(
        _tc1_body,
        grid=(_N // _R,),
        in_specs=[_row_spec(_D), _full_spec((_D, _D)), _cnt_spec()],
        out_specs=_row_spec(_D),
        out_shape=jax.ShapeDtypeStruct((_N, _D), jnp.float32),
    )(x, W1, counts)

    p1 = _agg128(src, dst, g1, z128)

    def mid(p, g, b, w, wout):
        return pl.pallas_call(
            _tc_mid_body,
            grid=(_N // _R,),
            in_specs=[
                pl.BlockSpec((_NSC, _R, _D), lambda i: (0, i, 0)),
                _row_spec(_D),
                _cnt_spec(),
                _full_spec((_D,)),
                _full_spec((_D, wout)),
            ],
            out_specs=_row_spec(wout),
            out_shape=jax.ShapeDtypeStruct((_N, wout), jnp.float32),
        )(p, g, counts, b, w)

    g2 = mid(p1, g1, b1, W2, _D)
    p2 = _agg128(src, dst, g2, z128)
    g3 = mid(p2, g2, b2, W3p, _CP)
    p3 = _agg128(src, dst, g3, z128)

    out = pl.pallas_call(
        _tc4_body,
        grid=(_N // _R,),
        in_specs=[
            pl.BlockSpec((_NSC, _R, _CP), lambda i: (0, i, 0)),
            _row_spec(_CP),
            _cnt_spec(),
            _full_spec((_CP,)),
        ],
        out_specs=_row_spec(_CP),
        out_shape=jax.ShapeDtypeStruct((_N, _CP), jnp.float32),
    )(p3, g3, counts, b3p)

    return out[:, :_C]
